# baseline (device time: 1417880 ns/iter reference)
import jax
import jax.numpy as jnp
from jax import lax
from jax.experimental import pallas as pl
from jax.experimental.pallas import tpu as pltpu

P = 16


def kernel(x, w_mat, scale_x, scale_w):
    m_full, _k_shard = x.shape
    _, n = w_mat.shape
    m_chunk = m_full // P

    def body(x_ref, w_ref, sx_ref, sw_ref, out_ref,
             comm_ref, send_sems, recv_sems, credit_sem):
        my = lax.axis_index("i")
        left = lax.rem(my + P - 1, P)
        right = lax.rem(my + 1, P)

        barrier = pltpu.get_barrier_semaphore()
        for nbr in (left, right):
            pl.semaphore_signal(barrier, inc=1, device_id=(nbr,),
                                device_id_type=pl.DeviceIdType.MESH)
        pl.semaphore_wait(barrier, 2)

        def partial(c):
            xc = x_ref[pl.ds(c * m_chunk, m_chunk), :]
            return lax.dot_general(
                xc, w_ref[:, :], (((1,), (0,)), ((), ())),
                preferred_element_type=jnp.float32)

        def send_desc(slot):
            return pltpu.make_async_remote_copy(
                src_ref=comm_ref.at[slot],
                dst_ref=comm_ref.at[1 - slot],
                send_sem=send_sems.at[slot],
                recv_sem=recv_sems.at[1 - slot],
                device_id=(right,),
                device_id_type=pl.DeviceIdType.MESH,
            )

        def recv_desc(slot):
            return pltpu.make_async_remote_copy(
                src_ref=comm_ref.at[slot],
                dst_ref=comm_ref.at[slot],
                send_sem=send_sems.at[slot],
                recv_sem=recv_sems.at[slot],
                device_id=(left,),
                device_id_type=pl.DeviceIdType.MESH,
            )

        c0 = lax.rem(my + P - 1, P)
        comm_ref[0] = partial(c0)
        s0 = send_desc(0)
        s0.start()
        s0.wait_send()

        for s in range(1, P):
            slot = s % 2
            c = lax.rem(my + 2 * P - 1 - s, P)
            recv_desc(slot).wait_recv()
            acc = comm_ref[slot] + partial(c)
            if s < P - 1:
                comm_ref[slot] = acc
                if s >= 2:
                    pl.semaphore_wait(credit_sem, 1)
                snd = send_desc(slot)
                snd.start()
                snd.wait_send()
                if s <= P - 3:
                    pl.semaphore_signal(credit_sem, inc=1, device_id=(left,),
                                        device_id_type=pl.DeviceIdType.MESH)
            else:
                out_ref[:, :] = acc * (sx_ref[0] * sw_ref[0])

    return pl.pallas_call(
        body,
        out_shape=jax.ShapeDtypeStruct((m_chunk, n), jnp.float32),
        in_specs=[
            pl.BlockSpec(memory_space=pltpu.VMEM),
            pl.BlockSpec(memory_space=pltpu.VMEM),
            pl.BlockSpec(memory_space=pltpu.SMEM),
            pl.BlockSpec(memory_space=pltpu.SMEM),
        ],
        out_specs=pl.BlockSpec(memory_space=pltpu.VMEM),
        scratch_shapes=[
            pltpu.VMEM((2, m_chunk, n), jnp.float32),
            pltpu.SemaphoreType.DMA((2,)),
            pltpu.SemaphoreType.DMA((2,)),
            pltpu.SemaphoreType.REGULAR,
        ],
        compiler_params=pltpu.CompilerParams(collective_id=0),
    )(x, w_mat, scale_x, scale_w)


# device time: 408674 ns/iter; 3.4695x vs baseline; 3.4695x over previous
import jax
import jax.numpy as jnp
from jax import lax
from jax.experimental import pallas as pl
from jax.experimental.pallas import tpu as pltpu

P = 16
COMM_DTYPE = jnp.bfloat16


def kernel(x, w_mat, scale_x, scale_w):
    m_full, _k_shard = x.shape
    _, n = w_mat.shape
    m_chunk = m_full // P
    nh = n // 2

    def body(x_ref, w_ref, sx_ref, sw_ref, out_ref,
             commA, commB, sendA, recvA, sendB, recvB, creditA, creditB):
        my = lax.axis_index("i")
        left = lax.rem(my + P - 1, P)
        right = lax.rem(my + 1, P)

        barrier = pltpu.get_barrier_semaphore()
        for nbr in (left, right):
            pl.semaphore_signal(barrier, inc=1, device_id=(nbr,),
                                device_id_type=pl.DeviceIdType.MESH)
        pl.semaphore_wait(barrier, 2)

        def partA(c):
            xc = x_ref[pl.ds(c * m_chunk, m_chunk), :]
            return lax.dot_general(
                xc, w_ref[:, :nh], (((1,), (0,)), ((), ())),
                preferred_element_type=jnp.float32)

        def partB(c):
            xc = x_ref[pl.ds(c * m_chunk, m_chunk), :]
            return lax.dot_general(
                xc, w_ref[:, nh:], (((1,), (0,)), ((), ())),
                preferred_element_type=jnp.float32)

        def desc(buf, src_slot, dst_slot, ssems, rsems, dev):
            return pltpu.make_async_remote_copy(
                src_ref=buf.at[src_slot],
                dst_ref=buf.at[dst_slot],
                send_sem=ssems.at[src_slot],
                recv_sem=rsems.at[dst_slot],
                device_id=(dev,),
                device_id_type=pl.DeviceIdType.MESH,
            )

        commA[0] = partA(lax.rem(my + P - 1, P)).astype(COMM_DTYPE)
        commB[0] = partB(lax.rem(my + 1, P)).astype(COMM_DTYPE)
        sA = desc(commA, 0, 1, sendA, recvA, right)
        sB = desc(commB, 0, 1, sendB, recvB, left)
        sA.start()
        sB.start()
        sA.wait_send()
        sB.wait_send()

        for s in range(1, P):
            slot = s % 2
            cA = lax.rem(my + 2 * P - 1 - s, P)
            cB = lax.rem(my + 1 + s, P)
            pa = partA(cA)
            pb = partB(cB)
            desc(commA, slot, slot, sendA, recvA, left).wait_recv()
            desc(commB, slot, slot, sendB, recvB, right).wait_recv()
            accA = commA[slot].astype(jnp.float32) + pa
            accB = commB[slot].astype(jnp.float32) + pb
            if s < P - 1:
                commA[slot] = accA.astype(COMM_DTYPE)
                commB[slot] = accB.astype(COMM_DTYPE)
                if s >= 2:
                    pl.semaphore_wait(creditA, 1)
                    pl.semaphore_wait(creditB, 1)
                sA = desc(commA, slot, 1 - slot, sendA, recvA, right)
                sB = desc(commB, slot, 1 - slot, sendB, recvB, left)
                sA.start()
                sB.start()
                sA.wait_send()
                sB.wait_send()
                if s <= P - 3:
                    pl.semaphore_signal(creditA, inc=1, device_id=(left,),
                                        device_id_type=pl.DeviceIdType.MESH)
                    pl.semaphore_signal(creditB, inc=1, device_id=(right,),
                                        device_id_type=pl.DeviceIdType.MESH)
            else:
                scale = sx_ref[0] * sw_ref[0]
                out_ref[:, :nh] = accA * scale
                out_ref[:, nh:] = accB * scale

    return pl.pallas_call(
        body,
        out_shape=jax.ShapeDtypeStruct((m_chunk, n), jnp.float32),
        in_specs=[
            pl.BlockSpec(memory_space=pltpu.VMEM),
            pl.BlockSpec(memory_space=pltpu.VMEM),
            pl.BlockSpec(memory_space=pltpu.SMEM),
            pl.BlockSpec(memory_space=pltpu.SMEM),
        ],
        out_specs=pl.BlockSpec(memory_space=pltpu.VMEM),
        scratch_shapes=[
            pltpu.VMEM((2, m_chunk, nh), COMM_DTYPE),
            pltpu.VMEM((2, m_chunk, nh), COMM_DTYPE),
            pltpu.SemaphoreType.DMA((2,)),
            pltpu.SemaphoreType.DMA((2,)),
            pltpu.SemaphoreType.DMA((2,)),
            pltpu.SemaphoreType.DMA((2,)),
            pltpu.SemaphoreType.REGULAR,
            pltpu.SemaphoreType.REGULAR,
        ],
        compiler_params=pltpu.CompilerParams(collective_id=0),
    )(x, w_mat, scale_x, scale_w)


# device time: 358519 ns/iter; 3.9548x vs baseline; 1.1399x over previous
import jax
import jax.numpy as jnp
from jax import lax
from jax.experimental import pallas as pl
from jax.experimental.pallas import tpu as pltpu

P = 16
COMM_DTYPE = jnp.bfloat16
NRINGS = 4


def kernel(x, w_mat, scale_x, scale_w):
    m_full, _k_shard = x.shape
    _, n = w_mat.shape
    m_chunk = m_full // P
    ns = n // NRINGS
    ring_cfg = [(True, 0), (True, ns), (False, 2 * ns), (False, 3 * ns)]

    def body(x_ref, w_ref, sx_ref, sw_ref, out_ref,
             comm, send_sems, recv_sems, credit_sems):
        my = lax.axis_index("i")
        left = lax.rem(my + P - 1, P)
        right = lax.rem(my + 1, P)

        barrier = pltpu.get_barrier_semaphore()
        for nbr in (left, right):
            pl.semaphore_signal(barrier, inc=1, device_id=(nbr,),
                                device_id_type=pl.DeviceIdType.MESH)
        pl.semaphore_wait(barrier, 2)

        scale = sx_ref[0] * sw_ref[0]

        def dots(s):
            cR = lax.rem(my + 2 * P - 1 - s, P)
            cL = lax.rem(my + 1 + s, P)
            xR = x_ref[pl.ds(cR * m_chunk, m_chunk), :]
            xL = x_ref[pl.ds(cL * m_chunk, m_chunk), :]
            out = []
            for ri, (rightward, c0) in enumerate(ring_cfg):
                xc = xR if rightward else xL
                out.append(lax.dot_general(
                    xc, w_ref[:, c0:c0 + ns], (((1,), (0,)), ((), ())),
                    preferred_element_type=jnp.float32))
            return out

        def desc(ri, src_slot, dst_slot, dev):
            return pltpu.make_async_remote_copy(
                src_ref=comm.at[ri, src_slot],
                dst_ref=comm.at[ri, dst_slot],
                send_sem=send_sems.at[ri, src_slot],
                recv_sem=recv_sems.at[ri, dst_slot],
                device_id=(dev,),
                device_id_type=pl.DeviceIdType.MESH,
            )

        sends = [None] * NRINGS
        for s in range(P):
            slot = s % 2
            pd = dots(s)
            for ri, (rightward, c0) in enumerate(ring_cfg):
                dst = right if rightward else left
                src = left if rightward else right
                if s >= 1:
                    sends[ri].wait_send()
                    if 1 <= s - 1 <= P - 3:
                        pl.semaphore_signal(
                            credit_sems.at[ri], inc=1, device_id=(src,),
                            device_id_type=pl.DeviceIdType.MESH)
                    desc(ri, slot, slot, src).wait_recv()
                    acc = comm[ri, slot].astype(jnp.float32) + pd[ri]
                else:
                    acc = pd[ri]
                if s < P - 1:
                    comm[ri, slot] = acc.astype(COMM_DTYPE)
                    if s >= 2:
                        pl.semaphore_wait(credit_sems.at[ri], 1)
                    snd = desc(ri, slot, 1 - slot, dst)
                    snd.start()
                    sends[ri] = snd
                else:
                    out_ref[:, c0:c0 + ns] = acc * scale

    return pl.pallas_call(
        body,
        out_shape=jax.ShapeDtypeStruct((m_chunk, n), jnp.float32),
        in_specs=[
            pl.BlockSpec(memory_space=pltpu.VMEM),
            pl.BlockSpec(memory_space=pltpu.VMEM),
            pl.BlockSpec(memory_space=pltpu.SMEM),
            pl.BlockSpec(memory_space=pltpu.SMEM),
        ],
        out_specs=pl.BlockSpec(memory_space=pltpu.VMEM),
        scratch_shapes=[
            pltpu.VMEM((NRINGS, 2, m_chunk, ns), COMM_DTYPE),
            pltpu.SemaphoreType.DMA((NRINGS, 2)),
            pltpu.SemaphoreType.DMA((NRINGS, 2)),
            pltpu.SemaphoreType.REGULAR((NRINGS,)),
        ],
        compiler_params=pltpu.CompilerParams(collective_id=0),
    )(x, w_mat, scale_x, scale_w)
